# initial kernel scaffold (unmeasured)
import jax
import jax.numpy as jnp
from jax import lax
from jax.experimental import pallas as pl
from jax.experimental.pallas import tpu as pltpu


def kernel(
    x,
):
    def body(*refs):
        pass

    out_shape = jax.ShapeDtypeStruct(..., jnp.float32)
    return pl.pallas_call(body, out_shape=out_shape)(...)


# baseline (device time: 20567 ns/iter reference)
import jax
import jax.numpy as jnp
from jax import lax
from jax.experimental import pallas as pl
from jax.experimental.pallas import tpu as pltpu

M = 1024
HALF = 512


def kernel(x):
    def body(x_ref, out_ref, ysend, yrecv, acc, send_sems, recv_sems):
        my_x = lax.axis_index("x")
        my_y = lax.axis_index("y")
        row0 = my_x * HALF
        col_mine0 = my_y * HALF
        col_peer0 = (1 - my_y) * HALF

        barrier_sem = pltpu.get_barrier_semaphore()
        pl.semaphore_signal(
            barrier_sem, inc=1,
            device_id=(my_x, 1 - my_y), device_id_type=pl.DeviceIdType.MESH,
        )
        pl.semaphore_signal(
            barrier_sem, inc=1,
            device_id=(1 - my_x, my_y), device_id_type=pl.DeviceIdType.MESH,
        )
        pl.semaphore_wait(barrier_sem, 2)

        ysend[...] = x_ref[
            0, pl.ds(row0, HALF), pl.ds(col_peer0, HALF)
        ].astype(jnp.bfloat16)
        rdma_y = pltpu.make_async_remote_copy(
            src_ref=ysend,
            dst_ref=yrecv,
            send_sem=send_sems.at[0],
            recv_sem=recv_sems.at[0],
            device_id=(my_x, 1 - my_y),
            device_id_type=pl.DeviceIdType.MESH,
        )
        rdma_y.start()

        acc[...] = x_ref[
            0, pl.ds(row0, HALF), pl.ds(col_mine0, HALF)
        ].astype(jnp.bfloat16)

        rdma_y.wait()
        acc[...] = acc[...] + yrecv[...]
        out_ref[pl.ds(row0, HALF), :] = acc[...]

        rdma_x = pltpu.make_async_remote_copy(
            src_ref=acc,
            dst_ref=out_ref.at[pl.ds(row0, HALF), :],
            send_sem=send_sems.at[1],
            recv_sem=recv_sems.at[1],
            device_id=(1 - my_x, my_y),
            device_id_type=pl.DeviceIdType.MESH,
        )
        rdma_x.start()
        rdma_x.wait()

    return pl.pallas_call(
        body,
        out_shape=jax.ShapeDtypeStruct((M, HALF), jnp.bfloat16),
        in_specs=[pl.BlockSpec(memory_space=pltpu.VMEM)],
        out_specs=pl.BlockSpec(memory_space=pltpu.VMEM),
        scratch_shapes=[
            pltpu.VMEM((HALF, HALF), jnp.bfloat16),
            pltpu.VMEM((HALF, HALF), jnp.bfloat16),
            pltpu.VMEM((HALF, HALF), jnp.bfloat16),
            pltpu.SemaphoreType.DMA((2,)),
            pltpu.SemaphoreType.DMA((2,)),
        ],
        compiler_params=pltpu.CompilerParams(collective_id=0),
    )(x)


# device time: 16388 ns/iter; 1.2550x vs baseline; 1.2550x over previous
import jax
import jax.numpy as jnp
from jax import lax
from jax.experimental import pallas as pl
from jax.experimental.pallas import tpu as pltpu

M = 1024
HALF = 512
C = 4
R = HALF // C


def kernel(x):
    def body(x_ref, out_ref, ysend, yrecv, acc, sem_ys, sem_yr, sem_xs, sem_xr):
        my_x = lax.axis_index("x")
        my_y = lax.axis_index("y")
        row0 = my_x * HALF
        col_mine0 = my_y * HALF
        col_peer0 = (1 - my_y) * HALF
        y_peer = (my_x, 1 - my_y)
        x_peer = (1 - my_x, my_y)

        barrier_sem = pltpu.get_barrier_semaphore()
        pl.semaphore_signal(
            barrier_sem, inc=1,
            device_id=y_peer, device_id_type=pl.DeviceIdType.MESH,
        )
        pl.semaphore_signal(
            barrier_sem, inc=1,
            device_id=x_peer, device_id_type=pl.DeviceIdType.MESH,
        )
        pl.semaphore_wait(barrier_sem, 2)

        rdmas_y = []
        for i in range(C):
            ysend[i] = x_ref[
                0, pl.ds(row0 + i * R, R), pl.ds(col_peer0, HALF)
            ].astype(jnp.bfloat16)
            r = pltpu.make_async_remote_copy(
                src_ref=ysend.at[i],
                dst_ref=yrecv.at[i],
                send_sem=sem_ys.at[i],
                recv_sem=sem_yr.at[i],
                device_id=y_peer,
                device_id_type=pl.DeviceIdType.MESH,
            )
            r.start()
            rdmas_y.append(r)

        for i in range(C):
            acc[i] = x_ref[
                0, pl.ds(row0 + i * R, R), pl.ds(col_mine0, HALF)
            ].astype(jnp.bfloat16)

        rdmas_x = []
        for i in range(C):
            rdmas_y[i].wait_recv()
            acc[i] = acc[i] + yrecv[i]
            r = pltpu.make_async_remote_copy(
                src_ref=acc.at[i],
                dst_ref=out_ref.at[pl.ds(row0 + i * R, R), :],
                send_sem=sem_xs.at[i],
                recv_sem=sem_xr.at[i],
                device_id=x_peer,
                device_id_type=pl.DeviceIdType.MESH,
            )
            r.start()
            rdmas_x.append(r)
            out_ref[pl.ds(row0 + i * R, R), :] = acc[i]

        for i in range(C):
            rdmas_y[i].wait_send()
            rdmas_x[i].wait_send()
            rdmas_x[i].wait_recv()

    return pl.pallas_call(
        body,
        out_shape=jax.ShapeDtypeStruct((M, HALF), jnp.bfloat16),
        in_specs=[pl.BlockSpec(memory_space=pltpu.VMEM)],
        out_specs=pl.BlockSpec(memory_space=pltpu.VMEM),
        scratch_shapes=[
            pltpu.VMEM((C, R, HALF), jnp.bfloat16),
            pltpu.VMEM((C, R, HALF), jnp.bfloat16),
            pltpu.VMEM((C, R, HALF), jnp.bfloat16),
            pltpu.SemaphoreType.DMA((C,)),
            pltpu.SemaphoreType.DMA((C,)),
            pltpu.SemaphoreType.DMA((C,)),
            pltpu.SemaphoreType.DMA((C,)),
        ],
        compiler_params=pltpu.CompilerParams(collective_id=0),
    )(x)
